# baseline (speedup 1.0000x reference)
import functools

import jax
import jax.numpy as jnp
from jax.experimental import pallas as pl
from jax.experimental.pallas import tpu as pltpu

T, D, E, RANK, B_DIM = 4096, 2048, 8, 16, 4096
ER = E * RANK
N_OUT = 2 * B_DIM

TM1 = 1024
TN2 = 128
NJH = B_DIM // TN2


def _xa_kernel(idx_ref, x_ref, ahall_ref, xa_ref):
    xa = jnp.dot(x_ref[...], ahall_ref[...], preferred_element_type=jnp.float32)
    col = jax.lax.broadcasted_iota(jnp.int32, (TM1, 2 * ER), 1)
    xa = jnp.where(((col // RANK) % E) == idx_ref[...], xa, 0.0)
    xa_ref[...] = jnp.stack([xa[:, :ER], xa[:, ER:]])


def _main_kernel(x_ref, w_ref, xa_ref, bres_ref, out_ref):
    h = pl.program_id(0) // NJH
    out_ref[...] = (
        jnp.dot(x_ref[...], w_ref[...], preferred_element_type=jnp.float32)
        + jnp.dot(xa_ref[h], bres_ref[...], preferred_element_type=jnp.float32)
    )


@functools.partial(jax.jit, static_argnames=())
def kernel(input_, W, A_buffer, B_buffer, weight_indices):
    A_hall = (A_buffer.reshape(E, D, 2, RANK)
              .transpose(1, 2, 0, 3).reshape(D, 2 * ER))
    B_res = B_buffer.reshape(ER, N_OUT)
    idx2d = weight_indices.astype(jnp.int32).reshape(T, 1)

    xa = pl.pallas_call(
        _xa_kernel,
        grid=(T // TM1,),
        in_specs=[
            pl.BlockSpec((TM1, 1), lambda i: (i, 0)),
            pl.BlockSpec((TM1, D), lambda i: (i, 0)),
            pl.BlockSpec((D, 2 * ER), lambda i: (0, 0)),
        ],
        out_specs=pl.BlockSpec((2, TM1, ER), lambda i: (0, i, 0)),
        out_shape=jax.ShapeDtypeStruct((2, T, ER), jnp.float32),
    )(idx2d, input_, A_hall)

    out = pl.pallas_call(
        _main_kernel,
        grid=(N_OUT // TN2,),
        in_specs=[
            pl.BlockSpec((T, D), lambda j: (0, 0)),
            pl.BlockSpec((D, TN2), lambda j: (0, j)),
            pl.BlockSpec((2, T, ER), lambda j: (0, 0, 0)),
            pl.BlockSpec((ER, TN2), lambda j: (0, j)),
        ],
        out_specs=pl.BlockSpec((T, TN2), lambda j: (0, j)),
        out_shape=jax.ShapeDtypeStruct((T, N_OUT), jnp.float32),
    )(input_, W, xa, B_res)
    return out


# speedup vs baseline: 1.7352x; 1.7352x over previous
import functools

import jax
import jax.numpy as jnp
from jax.experimental import pallas as pl
from jax.experimental.pallas import tpu as pltpu

T, D, E, RANK, B_DIM = 4096, 2048, 8, 16, 4096
ER = E * RANK
N_OUT = 2 * B_DIM

TM = 1024
TN = 1024
NJH = B_DIM // TN


def _fused_kernel(idx_ref, x_ref, w_ref, ahall_ref, bres_ref, out_ref, xa_ref):
    j = pl.program_id(1)

    @pl.when(j == 0)
    def _():
        xa = jnp.dot(x_ref[...], ahall_ref[...],
                     preferred_element_type=jnp.float32)
        col = jax.lax.broadcasted_iota(jnp.int32, (TM, 2 * ER), 1)
        col_expert = (col // RANK) % E
        xa = jnp.where(col_expert == idx_ref[...], xa, 0.0)
        xa_ref[0] = xa[:, :ER]
        xa_ref[1] = xa[:, ER:]

    h = j // NJH
    out_ref[...] = (
        jnp.dot(x_ref[...], w_ref[...], preferred_element_type=jnp.float32)
        + jnp.dot(xa_ref[h], bres_ref[...], preferred_element_type=jnp.float32)
    )


@functools.partial(jax.jit, static_argnames=())
def kernel(input_, W, A_buffer, B_buffer, weight_indices):
    A_hall = (A_buffer.reshape(E, D, 2, RANK)
              .transpose(1, 2, 0, 3).reshape(D, 2 * ER))
    B_res = B_buffer.reshape(ER, N_OUT)

    idx2d = weight_indices.astype(jnp.int32).reshape(T, 1)

    ni, nj = T // TM, N_OUT // TN
    out = pl.pallas_call(
        _fused_kernel,
        grid=(ni, nj),
        in_specs=[
            pl.BlockSpec((TM, 1), lambda i, j: (i, 0)),
            pl.BlockSpec((TM, D), lambda i, j: (i, 0)),
            pl.BlockSpec((D, TN), lambda i, j: (0, j)),
            pl.BlockSpec((D, 2 * ER), lambda i, j: (0, 0)),
            pl.BlockSpec((ER, TN), lambda i, j: (0, j)),
        ],
        out_specs=pl.BlockSpec((TM, TN), lambda i, j: (i, j)),
        out_shape=jax.ShapeDtypeStruct((T, N_OUT), jnp.float32),
        scratch_shapes=[pltpu.VMEM((2, TM, ER), jnp.float32)],
    )(idx2d, input_, W, A_hall, B_res)
    return out
